# baseline (speedup 1.0000x reference)
import jax
import jax.numpy as jnp
from jax import lax
from jax.experimental import pallas as pl
from jax.experimental.pallas import tpu as pltpu
from jax.experimental.pallas import tpu_sc as plsc

N = 10000
D = 128
E = 320000
L = 4
BN_EPS = 1e-5

NC = 2
NS = 16
NW = NC * NS

G = 128
STEPS = 80
K = 8
NWIN = STEPS // K
E_TILE = STEPS * G
E_PAD = NW * E_TILE

H_PAD = 10240
ROWS_PER_TILE = H_PAD // NS
DUMMY_ROW = N


def _sc_agg_body(h_hbm, src_hbm, dstw_hbm, zeros_hbm, out_hbm,
                 src_v, dstw_v, rows_v, agg_sh, wsem, gsem):
    c = lax.axis_index("c")
    s = lax.axis_index("s")

    pltpu.sync_copy(src_hbm.at[c, s], src_v)
    pltpu.sync_copy(dstw_hbm.at[c, s], dstw_v)

    pltpu.sync_copy(zeros_hbm,
                    agg_sh.at[pl.ds(s * ROWS_PER_TILE, ROWS_PER_TILE)])
    plsc.subcore_barrier()

    def step(j, carry):
        pltpu.async_copy(h_hbm.at[src_v.at[j]], rows_v,
                         gsem.at[0]).wait()
        pltpu.sync_copy(rows_v, agg_sh.at[dstw_v.at[j]], add=True)
        return carry

    lax.fori_loop(0, STEPS, step, 0, unroll=False)
    plsc.subcore_barrier()

    rows = pl.ds(s * ROWS_PER_TILE, ROWS_PER_TILE)
    pltpu.sync_copy(agg_sh.at[rows], out_hbm.at[c, rows])


def _sc_agg(h, src_t, dstw_t, zeros_blk):
    mesh = plsc.VectorSubcoreMesh(core_axis_name="c", subcore_axis_name="s")
    kern = pl.kernel(
        _sc_agg_body,
        out_type=jax.ShapeDtypeStruct((NC, H_PAD, D), jnp.float32),
        mesh=mesh,
        scratch_types=[
            pltpu.VMEM((STEPS, G), jnp.int32),
            pltpu.VMEM((STEPS, G), jnp.int32),
            pltpu.VMEM((G, D), jnp.float32),
            pltpu.VMEM_SHARED((H_PAD, D), jnp.float32),
            pltpu.SemaphoreType.DMA((2,)),
            pltpu.SemaphoreType.DMA((2,)),
        ],
    )
    return kern(h, src_t, dstw_t, zeros_blk)


def _tc_mlp_body(h_ref, agg_ref, w1_ref, b1_ref, w2_ref, b2_ref, o_ref):
    z = h_ref[...] + agg_ref[0] + agg_ref[1]
    z = jnp.dot(z, w1_ref[...], preferred_element_type=jnp.float32) + b1_ref[...]
    z = jnp.maximum(z, 0.0)
    z = jnp.dot(z, w2_ref[...], preferred_element_type=jnp.float32) + b2_ref[...]
    o_ref[...] = jnp.maximum(z, 0.0)


def _tc_mlp(h, aggp, w1f, b1f, w2, b2):
    B = 1024
    grid = (H_PAD // B,)
    return pl.pallas_call(
        _tc_mlp_body,
        grid=grid,
        in_specs=[
            pl.BlockSpec((B, D), lambda i: (i, 0)),
            pl.BlockSpec((NC, B, D), lambda i: (0, i, 0)),
            pl.BlockSpec((D, D), lambda i: (0, 0)),
            pl.BlockSpec((1, D), lambda i: (0, 0)),
            pl.BlockSpec((D, D), lambda i: (0, 0)),
            pl.BlockSpec((1, D), lambda i: (0, 0)),
        ],
        out_specs=pl.BlockSpec((B, D), lambda i: (i, 0)),
        out_shape=jax.ShapeDtypeStruct((H_PAD, D), jnp.float32),
    )(h, aggp, w1f, b1f, w2, b2)


def _tc_out_body(h_ref, w_ref, b_ref, o_ref):
    o_ref[...] = (
        jnp.dot(h_ref[...], w_ref[...], preferred_element_type=jnp.float32)
        + b_ref[...]
    )


def _tc_out(h, w_out, b_out):
    B = 1024
    grid = (H_PAD // B,)
    return pl.pallas_call(
        _tc_out_body,
        grid=grid,
        in_specs=[
            pl.BlockSpec((B, D), lambda i: (i, 0)),
            pl.BlockSpec((D, D), lambda i: (0, 0)),
            pl.BlockSpec((1, D), lambda i: (0, 0)),
        ],
        out_specs=pl.BlockSpec((B, D), lambda i: (i, 0)),
        out_shape=jax.ShapeDtypeStruct((H_PAD, D), jnp.float32),
    )(h, w_out, b_out)


def kernel(x, edge_index, W1, b1, gamma, beta, running_mean, running_var,
           W2, b2, W_out, b_out):
    src = edge_index[0]
    dst = edge_index[1]

    pad = E_PAD - E
    src_p = jnp.concatenate([src, jnp.zeros((pad,), jnp.int32)])
    dummy_dst = DUMMY_ROW + (jnp.arange(pad, dtype=jnp.int32) % (H_PAD - N))
    dst_p = jnp.concatenate([dst, dummy_dst])
    src_t = src_p.reshape(NC, NS, STEPS, G)
    dstw_t = dst_p.reshape(NC, NS, STEPS, G)

    scale = gamma * lax.rsqrt(running_var + BN_EPS)
    W1f = W1 * scale[:, None, :]
    b1f = (b1 - running_mean) * scale + beta

    h = jnp.pad(x, ((0, H_PAD - N), (0, 0)))
    zeros_blk = jnp.zeros((ROWS_PER_TILE, D), jnp.float32)

    for i in range(L):
        aggp = _sc_agg(h, src_t, dstw_t, zeros_blk)
        h = _tc_mlp(h, aggp, W1f[i], b1f[i][None, :], W2[i], b2[i][None, :])

    out = _tc_out(h, W_out, b_out[None, :])
    return out[:N]


# speedup vs baseline: 1.0013x; 1.0013x over previous
import jax
import jax.numpy as jnp
from jax import lax
from jax.experimental import pallas as pl
from jax.experimental.pallas import tpu as pltpu
from jax.experimental.pallas import tpu_sc as plsc

N = 10000
D = 128
E = 320000
L = 4
BN_EPS = 1e-5

NC = 2
NS = 16
NW = NC * NS

G = 128
STEPS = 80
K = 8
NWIN = STEPS // K
E_TILE = STEPS * G
E_PAD = NW * E_TILE

H_PAD = 10240
ROWS_PER_TILE = H_PAD // NS
DUMMY_ROW = N


def _sc_agg_body(h_hbm, src_hbm, dstw_hbm, zeros_hbm, out_hbm,
                 src_v, dstw_v, rows_v, agg_sh, gsem):
    c = lax.axis_index("c")
    s = lax.axis_index("s")

    pltpu.sync_copy(src_hbm.at[c, s], src_v)
    pltpu.sync_copy(dstw_hbm.at[c, s], dstw_v)

    pltpu.sync_copy(zeros_hbm,
                    agg_sh.at[pl.ds(s * ROWS_PER_TILE, ROWS_PER_TILE)])
    plsc.subcore_barrier()

    def step(j, carry):
        pltpu.async_copy(h_hbm.at[src_v.at[j]], rows_v, gsem).wait()
        pltpu.sync_copy(rows_v, agg_sh.at[dstw_v.at[j]], add=True)
        return carry

    lax.fori_loop(0, STEPS, step, 0, unroll=False)
    plsc.subcore_barrier()

    rows = pl.ds(s * ROWS_PER_TILE, ROWS_PER_TILE)
    pltpu.sync_copy(agg_sh.at[rows], out_hbm.at[c, rows])


def _sc_agg(h, src_t, dstw_t, zeros_blk):
    mesh = plsc.VectorSubcoreMesh(core_axis_name="c", subcore_axis_name="s")
    kern = pl.kernel(
        _sc_agg_body,
        out_type=jax.ShapeDtypeStruct((NC, H_PAD, D), jnp.float32),
        mesh=mesh,
        scratch_types=[
            pltpu.VMEM((STEPS, G), jnp.int32),
            pltpu.VMEM((STEPS, G), jnp.int32),
            pltpu.VMEM((G, D), jnp.float32),
            pltpu.VMEM_SHARED((H_PAD, D), jnp.float32),
            pltpu.SemaphoreType.DMA,
        ],
    )
    return kern(h, src_t, dstw_t, zeros_blk)


def _tc_mlp_body(h_ref, agg_ref, w1_ref, b1_ref, w2_ref, b2_ref, o_ref):
    z = h_ref[...] + agg_ref[0] + agg_ref[1]
    z = jnp.dot(z, w1_ref[...], preferred_element_type=jnp.float32) + b1_ref[...]
    z = jnp.maximum(z, 0.0)
    z = jnp.dot(z, w2_ref[...], preferred_element_type=jnp.float32) + b2_ref[...]
    o_ref[...] = jnp.maximum(z, 0.0)


def _tc_mlp(h, aggp, w1f, b1f, w2, b2):
    B = 1024
    grid = (H_PAD // B,)
    return pl.pallas_call(
        _tc_mlp_body,
        grid=grid,
        in_specs=[
            pl.BlockSpec((B, D), lambda i: (i, 0)),
            pl.BlockSpec((NC, B, D), lambda i: (0, i, 0)),
            pl.BlockSpec((D, D), lambda i: (0, 0)),
            pl.BlockSpec((1, D), lambda i: (0, 0)),
            pl.BlockSpec((D, D), lambda i: (0, 0)),
            pl.BlockSpec((1, D), lambda i: (0, 0)),
        ],
        out_specs=pl.BlockSpec((B, D), lambda i: (i, 0)),
        out_shape=jax.ShapeDtypeStruct((H_PAD, D), jnp.float32),
    )(h, aggp, w1f, b1f, w2, b2)


def _tc_out_body(h_ref, w_ref, b_ref, o_ref):
    o_ref[...] = (
        jnp.dot(h_ref[...], w_ref[...], preferred_element_type=jnp.float32)
        + b_ref[...]
    )


def _tc_out(h, w_out, b_out):
    B = 1024
    grid = (H_PAD // B,)
    return pl.pallas_call(
        _tc_out_body,
        grid=grid,
        in_specs=[
            pl.BlockSpec((B, D), lambda i: (i, 0)),
            pl.BlockSpec((D, D), lambda i: (0, 0)),
            pl.BlockSpec((1, D), lambda i: (0, 0)),
        ],
        out_specs=pl.BlockSpec((B, D), lambda i: (i, 0)),
        out_shape=jax.ShapeDtypeStruct((H_PAD, D), jnp.float32),
    )(h, w_out, b_out)


def kernel(x, edge_index, W1, b1, gamma, beta, running_mean, running_var,
           W2, b2, W_out, b_out):
    src = edge_index[0]
    dst = edge_index[1]

    pad = E_PAD - E
    src_p = jnp.concatenate([src, jnp.zeros((pad,), jnp.int32)])
    dst_p = jnp.concatenate([dst, jnp.full((pad,), DUMMY_ROW, jnp.int32)])
    src_t = src_p.reshape(NC, NS, STEPS, G)
    dstw_t = dst_p.reshape(NC, NS, STEPS, G)

    scale = gamma * lax.rsqrt(running_var + BN_EPS)
    W1f = W1 * scale[:, None, :]
    b1f = (b1 - running_mean) * scale + beta

    h = jnp.pad(x, ((0, H_PAD - N), (0, 0)))
    zeros_blk = jnp.zeros((ROWS_PER_TILE, D), jnp.float32)

    for i in range(L):
        aggp = _sc_agg(h, src_t, dstw_t, zeros_blk)
        h = _tc_mlp(h, aggp, W1f[i], b1f[i][None, :], W2[i], b2[i][None, :])

    out = _tc_out(h, W_out, b_out[None, :])
    return out[:N]


# speedup vs baseline: 3.0727x; 3.0686x over previous
import jax
import jax.numpy as jnp
from jax import lax
from jax.experimental import pallas as pl
from jax.experimental.pallas import tpu as pltpu
from jax.experimental.pallas import tpu_sc as plsc

N = 10000
D = 128
E = 320000
L = 4
BN_EPS = 1e-5

NC = 2
NS = 16
NW = NC * NS

G = 128
STEPS = 80
K = 8
NWIN = STEPS // K
E_TILE = STEPS * G
E_PAD = NW * E_TILE

H_PAD = 10240
ROWS_PER_TILE = H_PAD // NS
DUMMY_ROW = N


def _sc_agg_body(h_hbm, src_hbm, dstw_hbm, zeros_hbm, out_hbm,
                 src_v, dstw_v, rows_v, agg_sh, gsem):
    c = lax.axis_index("c")
    s = lax.axis_index("s")

    pltpu.sync_copy(src_hbm.at[c, s], src_v)
    pltpu.sync_copy(dstw_hbm.at[c, s], dstw_v)

    pltpu.sync_copy(zeros_hbm,
                    agg_sh.at[pl.ds(s * ROWS_PER_TILE, ROWS_PER_TILE)])
    plsc.subcore_barrier()

    def step(j, carry):
        pltpu.async_copy(h_hbm.at[src_v.at[j]], rows_v, gsem).wait()
        pltpu.sync_copy(rows_v, agg_sh.at[dstw_v.at[j]], add=True)
        return carry

    lax.fori_loop(0, STEPS, step, 0, unroll=False)
    plsc.subcore_barrier()

    rows = pl.ds(s * ROWS_PER_TILE, ROWS_PER_TILE)
    pltpu.sync_copy(agg_sh.at[rows], out_hbm.at[c, rows])


def _sc_agg(h, src_t, dstw_t, zeros_blk):
    mesh = plsc.VectorSubcoreMesh(core_axis_name="c", subcore_axis_name="s")
    kern = pl.kernel(
        _sc_agg_body,
        out_type=jax.ShapeDtypeStruct((NC, H_PAD, D), jnp.float32),
        mesh=mesh,
        scratch_types=[
            pltpu.VMEM((STEPS, G), jnp.int32),
            pltpu.VMEM((STEPS, G), jnp.int32),
            pltpu.VMEM((G, D), jnp.float32),
            pltpu.VMEM_SHARED((H_PAD, D), jnp.float32),
            pltpu.SemaphoreType.DMA,
        ],
    )
    return kern(h, src_t, dstw_t, zeros_blk)


def _tc_mlp_body(h_ref, agg_ref, w1_ref, b1_ref, w2_ref, b2_ref, o_ref):
    z = h_ref[...] + agg_ref[0] + agg_ref[1]
    z = jnp.dot(z, w1_ref[...], preferred_element_type=jnp.float32) + b1_ref[...]
    z = jnp.maximum(z, 0.0)
    z = jnp.dot(z, w2_ref[...], preferred_element_type=jnp.float32) + b2_ref[...]
    o_ref[...] = jnp.maximum(z, 0.0)


def _tc_mlp(h, aggp, w1f, b1f, w2, b2):
    B = 1024
    grid = (H_PAD // B,)
    return pl.pallas_call(
        _tc_mlp_body,
        grid=grid,
        in_specs=[
            pl.BlockSpec((B, D), lambda i: (i, 0)),
            pl.BlockSpec((NC, B, D), lambda i: (0, i, 0)),
            pl.BlockSpec((D, D), lambda i: (0, 0)),
            pl.BlockSpec((1, D), lambda i: (0, 0)),
            pl.BlockSpec((D, D), lambda i: (0, 0)),
            pl.BlockSpec((1, D), lambda i: (0, 0)),
        ],
        out_specs=pl.BlockSpec((B, D), lambda i: (i, 0)),
        out_shape=jax.ShapeDtypeStruct((H_PAD, D), jnp.float32),
    )(h, aggp, w1f, b1f, w2, b2)


def _tc_out_body(h_ref, w_ref, b_ref, o_ref):
    o_ref[...] = (
        jnp.dot(h_ref[...], w_ref[...], preferred_element_type=jnp.float32)
        + b_ref[...]
    )


def _tc_out(h, w_out, b_out):
    B = 1024
    grid = (H_PAD // B,)
    return pl.pallas_call(
        _tc_out_body,
        grid=grid,
        in_specs=[
            pl.BlockSpec((B, D), lambda i: (i, 0)),
            pl.BlockSpec((D, D), lambda i: (0, 0)),
            pl.BlockSpec((1, D), lambda i: (0, 0)),
        ],
        out_specs=pl.BlockSpec((B, D), lambda i: (i, 0)),
        out_shape=jax.ShapeDtypeStruct((H_PAD, D), jnp.float32),
    )(h, w_out, b_out)


def kernel(x, edge_index, W1, b1, gamma, beta, running_mean, running_var,
           W2, b2, W_out, b_out):
    src = edge_index[0]
    dst = edge_index[1]

    pad = E_PAD - E
    pad_src = jnp.arange(pad, dtype=jnp.int32) % N
    pad_dst = DUMMY_ROW + (jnp.arange(pad, dtype=jnp.int32) % (H_PAD - N))
    src_p = jnp.concatenate([src, pad_src])
    dst_p = jnp.concatenate([dst, pad_dst])
    src_t = src_p.reshape(NC, NS, STEPS, G)
    dstw_t = dst_p.reshape(NC, NS, STEPS, G)

    scale = gamma * lax.rsqrt(running_var + BN_EPS)
    W1f = W1 * scale[:, None, :]
    b1f = (b1 - running_mean) * scale + beta

    h = jnp.pad(x, ((0, H_PAD - N), (0, 0)))
    zeros_blk = jnp.zeros((ROWS_PER_TILE, D), jnp.float32)

    for i in range(L):
        aggp = _sc_agg(h, src_t, dstw_t, zeros_blk)
        h = _tc_mlp(h, aggp, W1f[i], b1f[i][None, :], W2[i], b2[i][None, :])

    out = _tc_out(h, W_out, b_out[None, :])
    return out[:N]
